# baseline (speedup 1.0000x reference)
import functools

import jax
import jax.numpy as jnp
from jax import lax
from jax.experimental import pallas as pl
from jax.experimental.pallas import tpu as pltpu, tpu_sc as plsc

BATCH = 16384
VOCAB = 1000000
D = 64
BLK = 16384
NBLK = 62
SLEN = NBLK * BLK
NC = 2
NS = 16
NW = NC * NS
BPW = BATCH // NW
NCHUNK = 4
CHUNK = BPW // NCHUNK


def _mv_body(tu_ref, tv_ref, wu_ref, wv_ref, su_ref, sv_ref):
    su = jax.lax.dot_general(
        wu_ref[...], tu_ref[...], (((0,), (0,)), ((), ())),
        preferred_element_type=jnp.float32,
    )
    sv = jax.lax.dot_general(
        wv_ref[...], tv_ref[...], (((0,), (0,)), ((), ())),
        preferred_element_type=jnp.float32,
    )
    su_ref[...] = su.reshape(BLK)
    sv_ref[...] = sv.reshape(BLK)


SC_COLS = 262144
SC_BLKOFF = SC_COLS // BLK
NBLK_TC = NBLK - SC_BLKOFF
CW = SC_COLS // NW
CCH = 256
NCHS = CW // CCH


def _scores(tu, tv, wu, wv):
    return pl.pallas_call(
        _mv_body,
        grid=(NBLK_TC,),
        in_specs=[
            pl.BlockSpec((D, BLK), lambda i: (0, i + SC_BLKOFF)),
            pl.BlockSpec((D, BLK), lambda i: (0, i + SC_BLKOFF)),
            pl.BlockSpec((D, 1), lambda i: (0, 0)),
            pl.BlockSpec((D, 1), lambda i: (0, 0)),
        ],
        out_specs=[
            pl.BlockSpec((BLK,), lambda i: (i,)),
            pl.BlockSpec((BLK,), lambda i: (i,)),
        ],
        out_shape=[
            jax.ShapeDtypeStruct((NBLK_TC * BLK,), jnp.float32),
            jax.ShapeDtypeStruct((NBLK_TC * BLK,), jnp.float32),
        ],
    )(tu, tv, wu, wv)


_mesh = plsc.VectorSubcoreMesh(
    core_axis_name="c", subcore_axis_name="s", num_cores=NC, num_subcores=NS
)


@functools.partial(
    pl.kernel,
    out_type=[
        jax.ShapeDtypeStruct((SC_COLS,), jnp.float32),
        jax.ShapeDtypeStruct((SC_COLS,), jnp.float32),
    ],
    mesh=_mesh,
    compiler_params=pltpu.CompilerParams(
        needs_layout_passes=False, use_tc_tiling_on_sc=True
    ),
    scratch_types=[
        pltpu.VMEM((D, CCH), jnp.float32),
        pltpu.VMEM((D, CCH), jnp.float32),
        pltpu.VMEM((D, CCH), jnp.float32),
        pltpu.VMEM((D, CCH), jnp.float32),
        pltpu.VMEM((128,), jnp.float32),
        pltpu.VMEM((CW,), jnp.float32),
        pltpu.VMEM((CW,), jnp.float32),
        pltpu.SemaphoreType.DMA,
        pltpu.SemaphoreType.DMA,
        pltpu.SemaphoreType.DMA,
        pltpu.SemaphoreType.DMA,
    ],
)
def _mv_front(tu_hbm, tv_hbm, w_hbm, su_hbm, sv_hbm,
              bu0, bu1, bi0, bi1, w_v, su_v, sv_v, smu0, smu1, smi0, smi1):
    wid = lax.axis_index("s") * NC + lax.axis_index("c")
    base = wid * CW

    pltpu.sync_copy(w_hbm, w_v)
    wch = [w_v[pl.ds(c * 16, 16)] for c in range(8)]

    def fire(ch, bu, bi, su, si):
        c0 = pl.multiple_of(base + ch * CCH, CCH)
        pltpu.async_copy(tu_hbm.at[:, pl.ds(c0, CCH)], bu, su)
        pltpu.async_copy(tv_hbm.at[:, pl.ds(c0, CCH)], bi, si)

    def drain(bu, bi, su, si):
        pltpu.make_async_copy(tu_hbm.at[:, pl.ds(0, CCH)], bu, su).wait()
        pltpu.make_async_copy(tv_hbm.at[:, pl.ds(0, CCH)], bi, si).wait()

    def compute(ch, bu, bi):
        def group(g, carry):
            accu = jnp.zeros((16,), jnp.float32)
            acci = jnp.zeros((16,), jnp.float32)
            for d in range(D):
                w_u = wch[d // 16][d % 16]
                w_i = wch[4 + d // 16][d % 16]
                accu = accu + bu[d, pl.ds(g * 16, 16)] * w_u
                acci = acci + bi[d, pl.ds(g * 16, 16)] * w_i
            su_v[pl.ds(ch * CCH + g * 16, 16)] = accu
            sv_v[pl.ds(ch * CCH + g * 16, 16)] = acci
            return carry

        lax.fori_loop(0, CCH // 16, group, 0)

    fire(0, bu0, bi0, smu0, smi0)
    fire(1, bu1, bi1, smu1, smi1)

    def pair(i, carry):
        drain(bu0, bi0, smu0, smi0)
        compute(2 * i, bu0, bi0)

        @pl.when(i < NCHS // 2 - 1)
        def _():
            fire(2 * i + 2, bu0, bi0, smu0, smi0)

        drain(bu1, bi1, smu1, smi1)
        compute(2 * i + 1, bu1, bi1)

        @pl.when(i < NCHS // 2 - 1)
        def _():
            fire(2 * i + 3, bu1, bi1, smu1, smi1)

        return carry

    lax.fori_loop(0, NCHS // 2, pair, 0)

    pltpu.sync_copy(su_v, su_hbm.at[pl.ds(base, CW)])
    pltpu.sync_copy(sv_v, sv_hbm.at[pl.ds(base, CW)])


@functools.partial(
    pl.kernel,
    out_type=jax.ShapeDtypeStruct((BATCH,), jnp.float32),
    mesh=_mesh,
    compiler_params=pltpu.CompilerParams(
        needs_layout_passes=False, use_tc_tiling_on_sc=False
    ),
    scratch_types=[
        pltpu.VMEM((NCHUNK, CHUNK), jnp.int32),
        pltpu.VMEM((NCHUNK, CHUNK), jnp.int32),
        pltpu.VMEM((NCHUNK, CHUNK), jnp.float32),
        pltpu.VMEM((NCHUNK, CHUNK), jnp.float32),
        pltpu.VMEM((16,), jnp.float32),
        pltpu.VMEM((BPW,), jnp.float32),
        pltpu.SemaphoreType.DMA,
    ],
)
def _gather_add(uid_hbm, mid_hbm, su_hbm, sv_hbm, bv_hbm, out_hbm,
                uid_v, mid_v, us_v, vs_v, bv_v, out_v, sem):
    wid = lax.axis_index("s") * NC + lax.axis_index("c")

    pltpu.sync_copy(bv_hbm, bv_v)
    pltpu.sync_copy(uid_hbm.at[wid], uid_v)
    pltpu.sync_copy(mid_hbm.at[wid], mid_v)

    copies = []
    for j in range(NCHUNK):
        copies.append(pltpu.async_copy(su_hbm.at[uid_v.at[j]], us_v.at[j], sem))
        copies.append(pltpu.async_copy(sv_hbm.at[mid_v.at[j]], vs_v.at[j], sem))
    for c in copies:
        c.wait()

    bvec = bv_v[pl.ds(0, 16)]

    def chunk_body(g, carry):
        j = g // (CHUNK // 16)
        kk = g - j * (CHUNK // 16)
        u16 = us_v[j, pl.ds(kk * 16, 16)]
        v16 = vs_v[j, pl.ds(kk * 16, 16)]
        out_v[pl.ds(g * 16, 16)] = u16 + v16 + bvec
        return carry

    lax.fori_loop(0, BPW // 16, chunk_body, 0)
    pltpu.sync_copy(out_v, out_hbm.at[pl.ds(wid * BPW, BPW)])


def kernel(user_id, movie_id, user_table, item_table, W, b):
    uid = user_id.astype(jnp.int32).reshape(NW, NCHUNK, CHUNK)
    mid = movie_id.astype(jnp.int32).reshape(NW, NCHUNK, CHUNK)
    wu = W[:D].reshape(D, 1)
    wv = W[D:].reshape(D, 1)
    bv = jnp.broadcast_to(b, (16,))
    tu = user_table.T
    tv = item_table.T
    su_f, sv_f = _mv_front(tu, tv, W.reshape(2 * D))
    su_m, sv_m = _scores(tu, tv, wu, wv)
    su = jnp.concatenate([su_f, su_m])
    sv = jnp.concatenate([sv_f, sv_m])
    out = _gather_add(uid, mid, su, sv, bv)
    return out.reshape(BATCH, 1)


# speedup vs baseline: 1.0721x; 1.0721x over previous
import functools

import jax
import jax.numpy as jnp
from jax import lax
from jax.experimental import pallas as pl
from jax.experimental.pallas import tpu as pltpu, tpu_sc as plsc

BATCH = 16384
VOCAB = 1000000
D = 64
BLK = 20480
NBLK = 49
SLEN = NBLK * BLK
NC = 2
NS = 16
NW = NC * NS
BPW = BATCH // NW
NCHUNK = 4
CHUNK = BPW // NCHUNK


def _mv_body(tu_ref, tv_ref, wu_ref, wv_ref, su_ref, sv_ref):
    su = jax.lax.dot_general(
        wu_ref[...], tu_ref[...], (((0,), (0,)), ((), ())),
        preferred_element_type=jnp.float32,
    )
    sv = jax.lax.dot_general(
        wv_ref[...], tv_ref[...], (((0,), (0,)), ((), ())),
        preferred_element_type=jnp.float32,
    )
    su_ref[...] = su.reshape(BLK)
    sv_ref[...] = sv.reshape(BLK)


def _scores(tu, tv, wu, wv):
    return pl.pallas_call(
        _mv_body,
        grid=(NBLK,),
        in_specs=[
            pl.BlockSpec((D, BLK), lambda i: (0, i)),
            pl.BlockSpec((D, BLK), lambda i: (0, i)),
            pl.BlockSpec((D, 1), lambda i: (0, 0)),
            pl.BlockSpec((D, 1), lambda i: (0, 0)),
        ],
        out_specs=[
            pl.BlockSpec((BLK,), lambda i: (i,)),
            pl.BlockSpec((BLK,), lambda i: (i,)),
        ],
        out_shape=[
            jax.ShapeDtypeStruct((SLEN,), jnp.float32),
            jax.ShapeDtypeStruct((SLEN,), jnp.float32),
        ],
    )(tu, tv, wu, wv)


_mesh = plsc.VectorSubcoreMesh(
    core_axis_name="c", subcore_axis_name="s", num_cores=NC, num_subcores=NS
)


@functools.partial(
    pl.kernel,
    out_type=jax.ShapeDtypeStruct((BATCH,), jnp.float32),
    mesh=_mesh,
    compiler_params=pltpu.CompilerParams(
        needs_layout_passes=False, use_tc_tiling_on_sc=False
    ),
    scratch_types=[
        pltpu.VMEM((NCHUNK, CHUNK), jnp.int32),
        pltpu.VMEM((NCHUNK, CHUNK), jnp.int32),
        pltpu.VMEM((NCHUNK, CHUNK), jnp.float32),
        pltpu.VMEM((NCHUNK, CHUNK), jnp.float32),
        pltpu.VMEM((16,), jnp.float32),
        pltpu.VMEM((BPW,), jnp.float32),
        pltpu.SemaphoreType.DMA,
    ],
)
def _gather_add(uid_hbm, mid_hbm, su_hbm, sv_hbm, bv_hbm, out_hbm,
                uid_v, mid_v, us_v, vs_v, bv_v, out_v, sem):
    wid = lax.axis_index("s") * NC + lax.axis_index("c")

    pltpu.sync_copy(bv_hbm, bv_v)
    pltpu.sync_copy(uid_hbm.at[wid], uid_v)
    pltpu.sync_copy(mid_hbm.at[wid], mid_v)

    copies = []
    for j in range(NCHUNK):
        copies.append(pltpu.async_copy(su_hbm.at[uid_v.at[j]], us_v.at[j], sem))
        copies.append(pltpu.async_copy(sv_hbm.at[mid_v.at[j]], vs_v.at[j], sem))
    for c in copies:
        c.wait()

    bvec = bv_v[pl.ds(0, 16)]

    def chunk_body(g, carry):
        j = g // (CHUNK // 16)
        kk = g - j * (CHUNK // 16)
        u16 = us_v[j, pl.ds(kk * 16, 16)]
        v16 = vs_v[j, pl.ds(kk * 16, 16)]
        out_v[pl.ds(g * 16, 16)] = u16 + v16 + bvec
        return carry

    lax.fori_loop(0, BPW // 16, chunk_body, 0)
    pltpu.sync_copy(out_v, out_hbm.at[pl.ds(wid * BPW, BPW)])


def kernel(user_id, movie_id, user_table, item_table, W, b):
    uid = user_id.astype(jnp.int32).reshape(NW, NCHUNK, CHUNK)
    mid = movie_id.astype(jnp.int32).reshape(NW, NCHUNK, CHUNK)
    wu = W[:D].reshape(D, 1)
    wv = W[D:].reshape(D, 1)
    bv = jnp.broadcast_to(b, (16,))
    su, sv = _scores(user_table.T, item_table.T, wu, wv)
    out = _gather_add(uid, mid, su, sv, bv)
    return out.reshape(BATCH, 1)
